# baseline (speedup 1.0000x reference)
import functools

import jax
import jax.numpy as jnp
from jax import lax
from jax.experimental import pallas as pl
from jax.experimental.pallas import tpu as pltpu

_TRUNK = 128
_D = 64
_PR = _TRUNK // 2
_W = 2 * _D
_NACC = 4


def _pass_a_kernel(k_ref, v_ref, m_ref, sp_ref, *, tb):
    b = pl.program_id(1)
    accs = [None] * _NACC
    for t in range(tb):
        sl = slice(t * _PR, (t + 1) * _PR)
        kp = k_ref[sl, :].astype(jnp.bfloat16)
        vp = v_ref[sl, :].astype(jnp.bfloat16)
        pf = lax.dot_general(
            kp, vp, (((0,), (0,)), ((), ())),
            preferred_element_type=jnp.float32)
        m_t = pf[:_D, :_D] + pf[_D:, _D:]
        m_ref[t] = m_t.astype(jnp.bfloat16)
        a = t % _NACC
        accs[a] = m_t if accs[a] is None else accs[a] + m_t
    s_blk = accs[0]
    for a in range(1, _NACC):
        s_blk = s_blk + accs[a]

    @pl.when(b == 0)
    def _():
        sp_ref[...] = jnp.zeros_like(sp_ref)

    sp_ref[...] += s_blk[None]


def _pass_b_kernel(q_ref, m_ref, sp_ref, o_ref, p_scr, *, tb):
    ssum = jnp.sum(sp_ref[...], axis=0)
    ssb = ssum.astype(jnp.bfloat16)
    mm = m_ref[...].reshape(tb * _D, _D)
    w_all = lax.dot_general(
        mm, ssb, (((1,), (1,)), ((), ())),
        preferred_element_type=jnp.float32)
    p_scr[...] = w_all.astype(jnp.bfloat16)
    zed = jnp.zeros((_D, _D), jnp.bfloat16)
    for t in range(tb):
        sl = slice(t * _PR, (t + 1) * _PR)
        qp = q_ref[sl, :].astype(jnp.bfloat16)
        wt = p_scr[t * _D:(t + 1) * _D, :]
        bd = jnp.concatenate(
            [jnp.concatenate([wt, zed], axis=1),
             jnp.concatenate([zed, wt], axis=1)], axis=0)
        o_ref[sl, :] = lax.dot_general(
            qp, bd, (((1,), (0,)), ((), ())),
            preferred_element_type=jnp.float32)


@jax.jit
def kernel(Q, K, V):
    N, D = Q.shape
    assert D == _D and N % _TRUNK == 0
    T = N // _TRUNK
    TB = 64
    P = 2
    assert T % TB == 0
    G = T // TB
    assert G % P == 0
    B1 = G // P
    RP = TB * _PR

    N2 = N // 2
    Q2 = Q.reshape(N2, _W)
    K2 = K.reshape(N2, _W)
    V2 = V.reshape(N2, _W)

    m_arr, s_parts = pl.pallas_call(
        functools.partial(_pass_a_kernel, tb=TB),
        grid=(P, B1),
        in_specs=[
            pl.BlockSpec((RP, _W), lambda p, b: (p * B1 + b, 0)),
            pl.BlockSpec((RP, _W), lambda p, b: (p * B1 + b, 0)),
        ],
        out_specs=[
            pl.BlockSpec((TB, _D, _D), lambda p, b: (p * B1 + b, 0, 0)),
            pl.BlockSpec((1, _D, _D), lambda p, b: (p, 0, 0)),
        ],
        out_shape=[
            jax.ShapeDtypeStruct((T, _D, _D), jnp.bfloat16),
            jax.ShapeDtypeStruct((P, _D, _D), jnp.float32),
        ],
        compiler_params=pltpu.CompilerParams(
            dimension_semantics=("parallel", "arbitrary"),
        ),
        name="la_pass_a",
    )(K2, V2)

    out2 = pl.pallas_call(
        functools.partial(_pass_b_kernel, tb=TB),
        grid=(P, B1),
        in_specs=[
            pl.BlockSpec((RP, _W), lambda p, b: (p * B1 + b, 0)),
            pl.BlockSpec((TB, _D, _D), lambda p, b: (p * B1 + b, 0, 0)),
            pl.BlockSpec((P, _D, _D), lambda p, b: (0, 0, 0)),
        ],
        out_specs=pl.BlockSpec((RP, _W), lambda p, b: (p * B1 + b, 0)),
        out_shape=jax.ShapeDtypeStruct((N2, _W), jnp.float32),
        scratch_shapes=[pltpu.VMEM((TB * _D, _D), jnp.bfloat16)],
        compiler_params=pltpu.CompilerParams(
            dimension_semantics=("parallel", "arbitrary"),
        ),
        name="la_pass_b",
    )(Q2, m_arr, s_parts)

    return out2.reshape(N, D)


# speedup vs baseline: 5.9309x; 5.9309x over previous
import functools

import jax
import jax.numpy as jnp
from jax import lax
from jax.experimental import pallas as pl
from jax.experimental.pallas import tpu as pltpu

_TRUNK = 128
_D = 64
_NACC = 4


def _pass_a_kernel(k_ref, v_ref, m_ref, sp_ref, *, tb):
    b = pl.program_id(1)
    accs = [None] * _NACC
    for t in range(tb):
        sl = slice(t * _TRUNK, (t + 1) * _TRUNK)
        kt = k_ref[:, sl].astype(jnp.bfloat16)
        vt = v_ref[:, sl].astype(jnp.bfloat16)
        m_t = lax.dot_general(
            kt, vt, (((1,), (1,)), ((), ())),
            preferred_element_type=jnp.float32)
        m_ref[t] = m_t.astype(jnp.bfloat16)
        a = t % _NACC
        accs[a] = m_t if accs[a] is None else accs[a] + m_t
    s_blk = accs[0]
    for a in range(1, _NACC):
        s_blk = s_blk + accs[a]

    @pl.when(b == 0)
    def _():
        sp_ref[...] = jnp.zeros_like(sp_ref)

    sp_ref[...] += s_blk[None]


def _pass_b_kernel(q_ref, m_ref, sp_ref, o_ref, p_scr, *, tb):
    ssum = jnp.sum(sp_ref[...], axis=0)
    ssb = ssum.astype(jnp.bfloat16)
    mm = m_ref[...].reshape(tb * _D, _D)
    w_all = lax.dot_general(
        mm, ssb, (((1,), (1,)), ((), ())),
        preferred_element_type=jnp.float32)
    p_scr[...] = w_all.astype(jnp.bfloat16)
    for t in range(tb):
        sl = slice(t * _TRUNK, (t + 1) * _TRUNK)
        qt = q_ref[:, sl].astype(jnp.bfloat16)
        wt = p_scr[t * _D:(t + 1) * _D, :]
        o_ref[:, sl] = lax.dot_general(
            wt, qt, (((0,), (0,)), ((), ())),
            preferred_element_type=jnp.float32)


@jax.jit
def kernel(Q, K, V):
    N, D = Q.shape
    assert D == _D and N % _TRUNK == 0
    T = N // _TRUNK
    TB = 64
    P = 2
    assert T % TB == 0
    G = T // TB
    assert G % P == 0
    B1 = G // P
    C = TB * _TRUNK

    QT = Q.T
    KT = K.T
    VT = V.T

    m_arr, s_parts = pl.pallas_call(
        functools.partial(_pass_a_kernel, tb=TB),
        grid=(P, B1),
        in_specs=[
            pl.BlockSpec((_D, C), lambda p, b: (0, p * B1 + b)),
            pl.BlockSpec((_D, C), lambda p, b: (0, p * B1 + b)),
        ],
        out_specs=[
            pl.BlockSpec((TB, _D, _D), lambda p, b: (p * B1 + b, 0, 0)),
            pl.BlockSpec((1, _D, _D), lambda p, b: (p, 0, 0)),
        ],
        out_shape=[
            jax.ShapeDtypeStruct((T, _D, _D), jnp.bfloat16),
            jax.ShapeDtypeStruct((P, _D, _D), jnp.float32),
        ],
        compiler_params=pltpu.CompilerParams(
            dimension_semantics=("parallel", "arbitrary"),
        ),
        name="la_pass_a",
    )(KT, VT)

    out_t = pl.pallas_call(
        functools.partial(_pass_b_kernel, tb=TB),
        grid=(P, B1),
        in_specs=[
            pl.BlockSpec((_D, C), lambda p, b: (0, p * B1 + b)),
            pl.BlockSpec((TB, _D, _D), lambda p, b: (p * B1 + b, 0, 0)),
            pl.BlockSpec((P, _D, _D), lambda p, b: (0, 0, 0)),
        ],
        out_specs=pl.BlockSpec((_D, C), lambda p, b: (0, p * B1 + b)),
        out_shape=jax.ShapeDtypeStruct((_D, N), jnp.float32),
        scratch_shapes=[pltpu.VMEM((TB * _D, _D), jnp.bfloat16)],
        compiler_params=pltpu.CompilerParams(
            dimension_semantics=("parallel", "arbitrary"),
        ),
        name="la_pass_b",
    )(QT, m_arr, s_parts)

    return out_t.T


# speedup vs baseline: 6.7474x; 1.1377x over previous
import functools

import jax
import jax.numpy as jnp
from jax import lax
from jax.experimental import pallas as pl
from jax.experimental.pallas import tpu as pltpu

_TRUNK = 128
_D = 64
_NACC = 4


def _fused_kernel(q_ref, k_ref, v_ref, o_ref, m_scr, s_scr, w_scr, *, tb):
    ph = pl.program_id(0)
    b = pl.program_id(1)

    @pl.when(ph == 0)
    def _phase0():
        accs = [None] * _NACC
        for t in range(tb):
            sl = slice(t * _TRUNK, (t + 1) * _TRUNK)
            kt = k_ref[:, sl].astype(jnp.bfloat16)
            vt = v_ref[:, sl].astype(jnp.bfloat16)
            m_t = lax.dot_general(
                kt, vt, (((1,), (1,)), ((), ())),
                preferred_element_type=jnp.float32)
            m_scr[pl.ds(b * tb + t, 1)] = m_t.astype(jnp.bfloat16)[None]
            a = t % _NACC
            accs[a] = m_t if accs[a] is None else accs[a] + m_t
        s_blk = accs[0]
        for a in range(1, _NACC):
            s_blk = s_blk + accs[a]

        @pl.when(b == 0)
        def _():
            s_scr[...] = jnp.zeros_like(s_scr)

        s_scr[...] += s_blk

    @pl.when(ph == 1)
    def _phase1():
        ssb = s_scr[...].astype(jnp.bfloat16)
        mm = m_scr[pl.ds(b * tb, tb)].reshape(tb * _D, _D)
        w_all = lax.dot_general(
            mm, ssb, (((1,), (1,)), ((), ())),
            preferred_element_type=jnp.float32)
        w_scr[...] = w_all.astype(jnp.bfloat16)
        for t in range(tb):
            sl = slice(t * _TRUNK, (t + 1) * _TRUNK)
            qt = q_ref[:, sl].astype(jnp.bfloat16)
            wt = w_scr[t * _D:(t + 1) * _D, :]
            o_ref[:, sl] = lax.dot_general(
                wt, qt, (((0,), (0,)), ((), ())),
                preferred_element_type=jnp.float32)


@jax.jit
def kernel(Q, K, V):
    N, D = Q.shape
    assert D == _D and N % _TRUNK == 0
    T = N // _TRUNK
    TB = 64
    assert T % TB == 0
    G = T // TB
    C = TB * _TRUNK

    QT = Q.T
    KT = K.T
    VT = V.T

    out_t = pl.pallas_call(
        functools.partial(_fused_kernel, tb=TB),
        grid=(2, G),
        in_specs=[
            pl.BlockSpec((_D, C), lambda ph, b: (0, b * ph)),
            pl.BlockSpec((_D, C), lambda ph, b: (0, b * (1 - ph))),
            pl.BlockSpec((_D, C), lambda ph, b: (0, b * (1 - ph))),
        ],
        out_specs=pl.BlockSpec((_D, C), lambda ph, b: (0, b * ph)),
        out_shape=jax.ShapeDtypeStruct((_D, N), jnp.float32),
        scratch_shapes=[
            pltpu.VMEM((T, _D, _D), jnp.bfloat16),
            pltpu.VMEM((_D, _D), jnp.float32),
            pltpu.VMEM((TB * _D, _D), jnp.bfloat16),
        ],
        compiler_params=pltpu.CompilerParams(
            dimension_semantics=("arbitrary", "arbitrary"),
            vmem_limit_bytes=56 * 1024 * 1024,
        ),
        name="la_fused",
    )(QT, KT, VT)

    return out_t.T


# speedup vs baseline: 8.0701x; 1.1960x over previous
import functools

import jax
import jax.numpy as jnp
from jax import lax
from jax.experimental import pallas as pl
from jax.experimental.pallas import tpu as pltpu

_TRUNK = 128
_D = 64
_NACC = 4


def _fused_kernel(q_ref, k_ref, v_ref, o_ref, m_scr, s_scr, w_scr, *, tb):
    ph = pl.program_id(0)
    b = pl.program_id(1)

    @pl.when(ph == 0)
    def _phase0():
        accs = [None] * _NACC
        for t2 in range(tb // 2):
            ms = []
            for half in range(2):
                t = 2 * t2 + half
                sl = slice(t * _TRUNK, (t + 1) * _TRUNK)
                kt = k_ref[:, sl].astype(jnp.bfloat16)
                vt = v_ref[:, sl].astype(jnp.bfloat16)
                m_t = lax.dot_general(
                    kt, vt, (((1,), (1,)), ((), ())),
                    preferred_element_type=jnp.float32)
                ms.append(m_t)
                a = t % _NACC
                accs[a] = m_t if accs[a] is None else accs[a] + m_t
            m_scr[pl.ds(b * (tb // 2) + t2, 1)] = jnp.concatenate(
                [ms[0].astype(jnp.bfloat16),
                 ms[1].astype(jnp.bfloat16)], axis=1)[None]
        s_blk = accs[0]
        for a in range(1, _NACC):
            s_blk = s_blk + accs[a]

        @pl.when(b == 0)
        def _():
            s_scr[...] = jnp.zeros_like(s_scr)

        s_scr[...] += s_blk

    @pl.when(ph == 1)
    def _phase1():
        ssb = s_scr[...].astype(jnp.bfloat16)
        zed = jnp.zeros((_D, _D), jnp.bfloat16)
        bd = jnp.concatenate(
            [jnp.concatenate([ssb, zed], axis=1),
             jnp.concatenate([zed, ssb], axis=1)], axis=0)
        mm = m_scr[pl.ds(b * (tb // 2), tb // 2)].reshape(tb // 2 * _D, 2 * _D)
        w_all = lax.dot_general(
            mm, bd, (((1,), (1,)), ((), ())),
            preferred_element_type=jnp.float32)
        w_scr[...] = w_all.astype(jnp.bfloat16)
        for t in range(tb):
            sl = slice(t * _TRUNK, (t + 1) * _TRUNK)
            qt = q_ref[:, sl].astype(jnp.bfloat16)
            t2, half = divmod(t, 2)
            wt = w_scr[t2 * _D:(t2 + 1) * _D,
                       half * _D:(half + 1) * _D]
            o_ref[:, sl] = lax.dot_general(
                wt, qt, (((0,), (0,)), ((), ())),
                preferred_element_type=jnp.float32)


@jax.jit
def kernel(Q, K, V):
    N, D = Q.shape
    assert D == _D and N % _TRUNK == 0
    T = N // _TRUNK
    TB = 128
    assert T % TB == 0 and TB % 2 == 0
    G = T // TB
    C = TB * _TRUNK

    QT = Q.T
    KT = K.T
    VT = V.T

    out_t = pl.pallas_call(
        functools.partial(_fused_kernel, tb=TB),
        grid=(2, G),
        in_specs=[
            pl.BlockSpec((_D, C), lambda ph, b: (0, b * ph)),
            pl.BlockSpec((_D, C), lambda ph, b: (0, b * (1 - ph))),
            pl.BlockSpec((_D, C), lambda ph, b: (0, b * (1 - ph))),
        ],
        out_specs=pl.BlockSpec((_D, C), lambda ph, b: (0, b * ph)),
        out_shape=jax.ShapeDtypeStruct((_D, N), jnp.float32),
        scratch_shapes=[
            pltpu.VMEM((T // 2, _D, 2 * _D), jnp.bfloat16),
            pltpu.VMEM((_D, _D), jnp.float32),
            pltpu.VMEM((TB // 2 * _D, 2 * _D), jnp.bfloat16),
        ],
        compiler_params=pltpu.CompilerParams(
            dimension_semantics=("arbitrary", "arbitrary"),
            vmem_limit_bytes=56 * 1024 * 1024,
        ),
        name="la_fused",
    )(QT, KT, VT)

    return out_t.T


# speedup vs baseline: 8.1700x; 1.0124x over previous
import functools

import jax
import jax.numpy as jnp
from jax import lax
from jax.experimental import pallas as pl
from jax.experimental.pallas import tpu as pltpu

_TRUNK = 128
_D = 64
_NACC = 4


def _fused_kernel(q_ref, k_ref, v_ref, o_ref, m_scr, s_scr, w_scr, *, tb):
    ph = pl.program_id(0)
    b = pl.program_id(1)

    @pl.when(ph == 0)
    def _phase0():
        accs = [None] * _NACC
        for t2 in range(tb // 2):
            ms = []
            for half in range(2):
                t = 2 * t2 + half
                sl = slice(t * _TRUNK, (t + 1) * _TRUNK)
                kt = k_ref[:, sl].astype(jnp.bfloat16)
                vt = v_ref[:, sl].astype(jnp.bfloat16)
                m_t = lax.dot_general(
                    kt, vt, (((1,), (1,)), ((), ())),
                    preferred_element_type=jnp.float32)
                ms.append(m_t)
                a = t % _NACC
                accs[a] = m_t if accs[a] is None else accs[a] + m_t
            m_scr[pl.ds(b * (tb // 2) + t2, 1)] = jnp.concatenate(
                [ms[0].astype(jnp.bfloat16),
                 ms[1].astype(jnp.bfloat16)], axis=1)[None]
        s_blk = accs[0]
        for a in range(1, _NACC):
            s_blk = s_blk + accs[a]

        @pl.when(b == 0)
        def _():
            s_scr[...] = jnp.zeros_like(s_scr)

        s_scr[...] += s_blk

    @pl.when(ph == 1)
    def _phase1():
        ssb = s_scr[...].astype(jnp.bfloat16)
        zed = jnp.zeros((_D, _D), jnp.bfloat16)
        bd = jnp.concatenate(
            [jnp.concatenate([ssb, zed], axis=1),
             jnp.concatenate([zed, ssb], axis=1)], axis=0)
        mm = m_scr[pl.ds(b * (tb // 2), tb // 2)].reshape(tb // 2 * _D, 2 * _D)
        w_all = lax.dot_general(
            mm, bd, (((1,), (1,)), ((), ())),
            preferred_element_type=jnp.float32)
        w_scr[...] = w_all.astype(jnp.bfloat16)
        for t in range(tb):
            sl = slice(t * _TRUNK, (t + 1) * _TRUNK)
            qt = q_ref[:, sl].astype(jnp.bfloat16)
            t2, half = divmod(t, 2)
            wt = w_scr[t2 * _D:(t2 + 1) * _D,
                       half * _D:(half + 1) * _D]
            o_ref[:, sl] = lax.dot_general(
                wt, qt, (((0,), (0,)), ((), ())),
                preferred_element_type=jnp.float32)


@jax.jit
def kernel(Q, K, V):
    N, D = Q.shape
    assert D == _D and N % _TRUNK == 0
    T = N // _TRUNK
    TB = 128
    assert T % TB == 0 and TB % 2 == 0
    G = T // TB
    C = TB * _TRUNK

    QT = Q.T
    KT = K.T
    VT = V.T

    out_t = pl.pallas_call(
        functools.partial(_fused_kernel, tb=TB),
        grid=(2, G),
        in_specs=[
            pl.BlockSpec((_D, C), lambda ph, b: (0, b * ph)),
            pl.BlockSpec((_D, C), lambda ph, b: (0, b + ph * (G - 1 - b))),
            pl.BlockSpec((_D, C), lambda ph, b: (0, b + ph * (G - 1 - b))),
        ],
        out_specs=pl.BlockSpec((_D, C), lambda ph, b: (0, b * ph)),
        out_shape=jax.ShapeDtypeStruct((_D, N), jnp.float32),
        scratch_shapes=[
            pltpu.VMEM((T // 2, _D, 2 * _D), jnp.bfloat16),
            pltpu.VMEM((_D, _D), jnp.float32),
            pltpu.VMEM((TB // 2 * _D, 2 * _D), jnp.bfloat16),
        ],
        compiler_params=pltpu.CompilerParams(
            dimension_semantics=("arbitrary", "arbitrary"),
            vmem_limit_bytes=56 * 1024 * 1024,
        ),
        name="la_fused",
    )(QT, KT, VT)

    return out_t.T
